# baseline (speedup 1.0000x reference)
import functools

import jax
import jax.numpy as jnp
from jax import lax
from jax.experimental import pallas as pl
from jax.experimental.pallas import tpu as pltpu
from jax.experimental.pallas import tpu_sc as plsc

N = 10000
E = 320000
D = 128
BN_EPS = 1e-5

NUM_CORES = 2
NUM_SUBCORES = 16
COLS = D // NUM_CORES
CHUNK = 128
CHUNKS_PER_T = 160
NSEG = 4
SEG_CHUNKS = CHUNKS_PER_T // NSEG
SEG_EDGES = SEG_CHUNKS * CHUNK
E_PAD = NUM_SUBCORES * CHUNK * CHUNKS_PER_T
N_ROWS = 10112
ROWS_PER_TILE = N_ROWS // NUM_SUBCORES


def _sc_body(vcs_hbm, srcp_hbm, dstp_hbm, zeros_hbm, out_hbm,
             v_sp, acc, src_idx, dst_idx, rows):
    cid = lax.axis_index("c")
    sid = lax.axis_index("s")
    rslice = pl.ds(sid * ROWS_PER_TILE, ROWS_PER_TILE)

    pltpu.sync_copy(vcs_hbm.at[cid, rslice], v_sp.at[rslice])
    pltpu.sync_copy(zeros_hbm, acc.at[rslice])
    plsc.subcore_barrier()

    for s in range(NSEG):
        pltpu.sync_copy(srcp_hbm.at[sid, pl.ds(s * SEG_EDGES, SEG_EDGES)],
                        src_idx)
        pltpu.sync_copy(dstp_hbm.at[sid, pl.ds(s * SEG_CHUNKS, SEG_CHUNKS)],
                        dst_idx)

        def body(j, carry):
            pltpu.sync_copy(v_sp.at[src_idx.at[pl.ds(j * 2 * CHUNK,
                                                     2 * CHUNK)]], rows)
            for i in range(2):
                pltpu.sync_copy(rows.at[pl.ds(i * CHUNK, CHUNK)],
                                acc.at[dst_idx.at[j * 2 + i]], add=True)
            return carry

        lax.fori_loop(0, SEG_CHUNKS // 2, body, 0)
    plsc.subcore_barrier()
    pltpu.sync_copy(acc.at[rslice], out_hbm.at[cid, rslice])


_sc_aggregate = functools.partial(
    pl.kernel,
    out_type=jax.ShapeDtypeStruct((NUM_CORES, N_ROWS, COLS), jnp.float32),
    mesh=plsc.VectorSubcoreMesh(
        core_axis_name="c", subcore_axis_name="s",
        num_cores=NUM_CORES, num_subcores=NUM_SUBCORES),
    scratch_types=[
        pltpu.VMEM_SHARED((N_ROWS, COLS), jnp.float32),
        pltpu.VMEM_SHARED((N_ROWS, COLS), jnp.float32),
        pltpu.VMEM((SEG_EDGES,), jnp.int32),
        pltpu.VMEM((SEG_CHUNKS, CHUNK), jnp.int32),
        pltpu.VMEM((2 * CHUNK, COLS), jnp.float32),
    ],
)(_sc_body)


def _tc_body(p_ref, v_ref, eps_ref, W1_ref, b1_ref, g1_ref, be1_ref,
             W2_ref, b2_ref, g2_ref, be2_ref, out_ref):
    eps = eps_ref[0, 0]
    x = jnp.concatenate([p_ref[0, :N, :], p_ref[1, :N, :]], axis=1)
    x = x + eps * v_ref[...]

    h = lax.dot_general(x, W1_ref[...], (((1,), (1,)), ((), ())),
                        preferred_element_type=jnp.float32) + b1_ref[...]
    mean = jnp.mean(h, axis=0, keepdims=True)
    var = jnp.mean((h - mean) * (h - mean), axis=0, keepdims=True)
    h = (h - mean) * lax.rsqrt(var + BN_EPS) * g1_ref[...] + be1_ref[...]
    h = jnp.maximum(h, 0.0)

    h = lax.dot_general(h, W2_ref[...], (((1,), (1,)), ((), ())),
                        preferred_element_type=jnp.float32) + b2_ref[...]
    mean = jnp.mean(h, axis=0, keepdims=True)
    var = jnp.mean((h - mean) * (h - mean), axis=0, keepdims=True)
    h = (h - mean) * lax.rsqrt(var + BN_EPS) * g2_ref[...] + be2_ref[...]
    out_ref[...] = jnp.maximum(h, 0.0)


def kernel(v, a, epsilon, W1, b1, g1, be1, W2, b2, g2, be2):
    src = a[0].astype(jnp.int32)
    dst = a[1].astype(jnp.int32)
    pad = E_PAD - E
    srcp = jnp.concatenate([src, jnp.full((pad,), N, jnp.int32)])
    dstp = jnp.concatenate([dst, jnp.zeros((pad,), jnp.int32)])
    srcp = srcp.reshape(NUM_SUBCORES, CHUNKS_PER_T * CHUNK)
    dstp = dstp.reshape(NUM_SUBCORES, CHUNKS_PER_T, CHUNK)
    vp = jnp.concatenate([v, jnp.zeros((N_ROWS - N, D), jnp.float32)])
    vcs = jnp.stack([vp[:, :COLS], vp[:, COLS:]])
    zeros_blk = jnp.zeros((ROWS_PER_TILE, COLS), jnp.float32)

    parts = _sc_aggregate(vcs, srcp, dstp, zeros_blk)

    out = pl.pallas_call(
        _tc_body,
        out_shape=jax.ShapeDtypeStruct((N, D), jnp.float32),
    )(parts, v, epsilon,
      W1, b1.reshape(1, D), g1.reshape(1, D), be1.reshape(1, D),
      W2, b2.reshape(1, D), g2.reshape(1, D), be2.reshape(1, D))
    return out


# speedup vs baseline: 1.2372x; 1.2372x over previous
import functools

import jax
import jax.numpy as jnp
from jax import lax
from jax.experimental import pallas as pl
from jax.experimental.pallas import tpu as pltpu
from jax.experimental.pallas import tpu_sc as plsc

N = 10000
E = 320000
D = 128
BN_EPS = 1e-5

NUM_CORES = 2
NUM_SUBCORES = 16
COLS = D // NUM_CORES
CHUNK = 128
CHUNKS_PER_T = 160
NSEG = 4
SEG_CHUNKS = CHUNKS_PER_T // NSEG
BLK = 8
E_PAD = NUM_SUBCORES * CHUNK * CHUNKS_PER_T
N_ROWS = 10112
ROWS_PER_TILE = N_ROWS // NUM_SUBCORES


def _sc_body(vcs_hbm, srcp_hbm, dstp_hbm, zeros_hbm, out_hbm,
             v_sp, acc, src_idx, dst_idx, rows, gsem, ssem):
    cid = lax.axis_index("c")
    sid = lax.axis_index("s")
    rslice = pl.ds(sid * ROWS_PER_TILE, ROWS_PER_TILE)

    pltpu.sync_copy(vcs_hbm.at[cid, rslice], v_sp.at[rslice])
    pltpu.sync_copy(zeros_hbm, acc.at[rslice])
    plsc.subcore_barrier()

    for s in range(NSEG):
        pltpu.sync_copy(srcp_hbm.at[sid, pl.ds(s * SEG_CHUNKS, SEG_CHUNKS)],
                        src_idx)
        pltpu.sync_copy(dstp_hbm.at[sid, pl.ds(s * SEG_CHUNKS, SEG_CHUNKS)],
                        dst_idx)

        def body(j, carry):
            base = j * BLK
            d_g = {}
            d_s = {}
            d_g[0] = pltpu.async_copy(
                v_sp.at[src_idx.at[base]], rows.at[0], gsem)
            for i in range(BLK):
                if i + 1 < BLK:
                    if i >= 1:
                        d_s[i - 1].wait()
                    d_g[i + 1] = pltpu.async_copy(
                        v_sp.at[src_idx.at[base + i + 1]],
                        rows.at[(i + 1) % 2], gsem)
                d_g[i].wait()
                d_s[i] = pltpu.async_copy(
                    rows.at[i % 2], acc.at[dst_idx.at[base + i]], ssem,
                    add=True)
            d_s[BLK - 2].wait()
            d_s[BLK - 1].wait()
            return carry

        lax.fori_loop(0, SEG_CHUNKS // BLK, body, 0)
    plsc.subcore_barrier()
    pltpu.sync_copy(acc.at[rslice], out_hbm.at[cid, rslice])


_sc_aggregate = functools.partial(
    pl.kernel,
    out_type=jax.ShapeDtypeStruct((NUM_CORES, N_ROWS, COLS), jnp.float32),
    mesh=plsc.VectorSubcoreMesh(
        core_axis_name="c", subcore_axis_name="s",
        num_cores=NUM_CORES, num_subcores=NUM_SUBCORES),
    scratch_types=[
        pltpu.VMEM_SHARED((N_ROWS, COLS), jnp.float32),
        pltpu.VMEM_SHARED((N_ROWS, COLS), jnp.float32),
        pltpu.VMEM((SEG_CHUNKS, CHUNK), jnp.int32),
        pltpu.VMEM((SEG_CHUNKS, CHUNK), jnp.int32),
        pltpu.VMEM((2, CHUNK, COLS), jnp.float32),
        pltpu.SemaphoreType.DMA,
        pltpu.SemaphoreType.DMA,
    ],
)(_sc_body)


def _tc_body(p_ref, v_ref, eps_ref, W1_ref, b1_ref, g1_ref, be1_ref,
             W2_ref, b2_ref, g2_ref, be2_ref, out_ref):
    eps = eps_ref[0, 0]
    x = jnp.concatenate([p_ref[0, :N, :], p_ref[1, :N, :]], axis=1)
    x = x + eps * v_ref[...]

    h = lax.dot_general(x, W1_ref[...], (((1,), (1,)), ((), ())),
                        preferred_element_type=jnp.float32) + b1_ref[...]
    mean = jnp.mean(h, axis=0, keepdims=True)
    var = jnp.mean((h - mean) * (h - mean), axis=0, keepdims=True)
    h = (h - mean) * lax.rsqrt(var + BN_EPS) * g1_ref[...] + be1_ref[...]
    h = jnp.maximum(h, 0.0)

    h = lax.dot_general(h, W2_ref[...], (((1,), (1,)), ((), ())),
                        preferred_element_type=jnp.float32) + b2_ref[...]
    mean = jnp.mean(h, axis=0, keepdims=True)
    var = jnp.mean((h - mean) * (h - mean), axis=0, keepdims=True)
    h = (h - mean) * lax.rsqrt(var + BN_EPS) * g2_ref[...] + be2_ref[...]
    out_ref[...] = jnp.maximum(h, 0.0)


def kernel(v, a, epsilon, W1, b1, g1, be1, W2, b2, g2, be2):
    src = a[0].astype(jnp.int32)
    dst = a[1].astype(jnp.int32)
    pad = E_PAD - E
    srcp = jnp.concatenate([src, jnp.full((pad,), N, jnp.int32)])
    dstp = jnp.concatenate([dst, jnp.zeros((pad,), jnp.int32)])
    srcp = srcp.reshape(NUM_SUBCORES, CHUNKS_PER_T, CHUNK)
    dstp = dstp.reshape(NUM_SUBCORES, CHUNKS_PER_T, CHUNK)
    vp = jnp.concatenate([v, jnp.zeros((N_ROWS - N, D), jnp.float32)])
    vcs = jnp.stack([vp[:, :COLS], vp[:, COLS:]])
    zeros_blk = jnp.zeros((ROWS_PER_TILE, COLS), jnp.float32)

    parts = _sc_aggregate(vcs, srcp, dstp, zeros_blk)

    out = pl.pallas_call(
        _tc_body,
        out_shape=jax.ShapeDtypeStruct((N, D), jnp.float32),
    )(parts, v, epsilon,
      W1, b1.reshape(1, D), g1.reshape(1, D), be1.reshape(1, D),
      W2, b2.reshape(1, D), g2.reshape(1, D), be2.reshape(1, D))
    return out


# speedup vs baseline: 1.2588x; 1.0175x over previous
import functools

import jax
import jax.numpy as jnp
from jax import lax
from jax.experimental import pallas as pl
from jax.experimental.pallas import tpu as pltpu
from jax.experimental.pallas import tpu_sc as plsc

N = 10000
E = 320000
D = 128
BN_EPS = 1e-5

NUM_CORES = 2
NUM_SUBCORES = 16
COLS = D // NUM_CORES
CHUNK = 128
CHUNKS_PER_T = 160
NSEG = 4
SEG_CHUNKS = CHUNKS_PER_T // NSEG
BLK = 8
E_PAD = NUM_SUBCORES * CHUNK * CHUNKS_PER_T
N_ROWS = 10112
ROWS_PER_TILE = N_ROWS // NUM_SUBCORES


def _sc_body(vcs_hbm, srcp_hbm, dstp_hbm, zeros_hbm, out_hbm,
             v_sp, acc, src_idx, dst_idx, rows, gsem, ssem):
    cid = lax.axis_index("c")
    sid = lax.axis_index("s")
    rslice = pl.ds(sid * ROWS_PER_TILE, ROWS_PER_TILE)

    pltpu.sync_copy(vcs_hbm.at[cid, rslice], v_sp.at[rslice])
    pltpu.sync_copy(zeros_hbm, acc.at[rslice])
    plsc.subcore_barrier()

    for s in range(NSEG):
        pltpu.sync_copy(srcp_hbm.at[sid, pl.ds(s * SEG_CHUNKS, SEG_CHUNKS)],
                        src_idx)
        pltpu.sync_copy(dstp_hbm.at[sid, pl.ds(s * SEG_CHUNKS, SEG_CHUNKS)],
                        dst_idx)

        def body(j, carry):
            base = j * BLK
            d_g = {}
            d_s = {}
            d_g[0] = pltpu.async_copy(
                v_sp.at[src_idx.at[base]], rows.at[0], gsem)
            for i in range(BLK):
                if i + 1 < BLK:
                    if i >= 1:
                        d_s[i - 1].wait()
                    d_g[i + 1] = pltpu.async_copy(
                        v_sp.at[src_idx.at[base + i + 1]],
                        rows.at[(i + 1) % 2], gsem)
                d_g[i].wait()
                d_s[i] = pltpu.async_copy(
                    rows.at[i % 2], acc.at[dst_idx.at[base + i]], ssem,
                    add=True)
            d_s[BLK - 2].wait()
            d_s[BLK - 1].wait()
            return carry

        lax.fori_loop(0, SEG_CHUNKS // BLK, body, 0)
    plsc.subcore_barrier()
    pltpu.sync_copy(acc.at[rslice], out_hbm.at[cid, rslice])


_sc_aggregate = functools.partial(
    pl.kernel,
    out_type=jax.ShapeDtypeStruct((NUM_CORES, N_ROWS, COLS), jnp.float32),
    mesh=plsc.VectorSubcoreMesh(
        core_axis_name="c", subcore_axis_name="s",
        num_cores=NUM_CORES, num_subcores=NUM_SUBCORES),
    scratch_types=[
        pltpu.VMEM_SHARED((N_ROWS, COLS), jnp.float32),
        pltpu.VMEM_SHARED((N_ROWS, COLS), jnp.float32),
        pltpu.VMEM((SEG_CHUNKS, CHUNK), jnp.int32),
        pltpu.VMEM((SEG_CHUNKS, CHUNK), jnp.int32),
        pltpu.VMEM((2, CHUNK, COLS), jnp.float32),
        pltpu.SemaphoreType.DMA,
        pltpu.SemaphoreType.DMA,
    ],
)(_sc_body)


def _tc_body(p_ref, v_ref, eps_ref, W1_ref, b1_ref, g1_ref, be1_ref,
             W2_ref, b2_ref, g2_ref, be2_ref, out_ref):
    eps = eps_ref[0, 0]
    x = jnp.concatenate([p_ref[0, :N, :], p_ref[1, :N, :]], axis=1)
    x = x + eps * v_ref[...]

    h = lax.dot_general(x, W1_ref[...], (((1,), (1,)), ((), ())),
                        preferred_element_type=jnp.float32) + b1_ref[...]
    mean = jnp.mean(h, axis=0, keepdims=True)
    var = jnp.mean((h - mean) * (h - mean), axis=0, keepdims=True)
    h = (h - mean) * lax.rsqrt(var + BN_EPS) * g1_ref[...] + be1_ref[...]
    h = jnp.maximum(h, 0.0)

    h = lax.dot_general(h, W2_ref[...], (((1,), (1,)), ((), ())),
                        preferred_element_type=jnp.float32) + b2_ref[...]
    mean = jnp.mean(h, axis=0, keepdims=True)
    var = jnp.mean((h - mean) * (h - mean), axis=0, keepdims=True)
    h = (h - mean) * lax.rsqrt(var + BN_EPS) * g2_ref[...] + be2_ref[...]
    out_ref[...] = jnp.maximum(h, 0.0)


def kernel(v, a, epsilon, W1, b1, g1, be1, W2, b2, g2, be2):
    src = a[0].astype(jnp.int32)
    dst = a[1].astype(jnp.int32)
    pad = E_PAD - E
    spread = N + (jnp.arange(pad, dtype=jnp.int32) % (N_ROWS - N))
    srcp = jnp.concatenate([src, spread])
    dstp = jnp.concatenate([dst, spread])
    srcp = srcp.reshape(NUM_SUBCORES, CHUNKS_PER_T, CHUNK)
    dstp = dstp.reshape(NUM_SUBCORES, CHUNKS_PER_T, CHUNK)
    vp = jnp.concatenate([v, jnp.zeros((N_ROWS - N, D), jnp.float32)])
    vcs = jnp.stack([vp[:, :COLS], vp[:, COLS:]])
    zeros_blk = jnp.zeros((ROWS_PER_TILE, COLS), jnp.float32)

    parts = _sc_aggregate(vcs, srcp, dstp, zeros_blk)

    out = pl.pallas_call(
        _tc_body,
        out_shape=jax.ShapeDtypeStruct((N, D), jnp.float32),
    )(parts, v, epsilon,
      W1, b1.reshape(1, D), g1.reshape(1, D), be1.reshape(1, D),
      W2, b2.reshape(1, D), g2.reshape(1, D), be2.reshape(1, D))
    return out


# speedup vs baseline: 1.2605x; 1.0013x over previous
import functools

import jax
import jax.numpy as jnp
from jax import lax
from jax.experimental import pallas as pl
from jax.experimental.pallas import tpu as pltpu
from jax.experimental.pallas import tpu_sc as plsc

N = 10000
E = 320000
D = 128
BN_EPS = 1e-5

NUM_CORES = 2
NUM_SUBCORES = 16
COLS = D // NUM_CORES
CHUNK = 128
CHUNKS_PER_T = 160
NSEG = 4
SEG_CHUNKS = CHUNKS_PER_T // NSEG
BLK = 8
NBUF = 2
E_PAD = NUM_SUBCORES * CHUNK * CHUNKS_PER_T
N_ROWS = 10112
ROWS_PER_TILE = N_ROWS // NUM_SUBCORES


def _sc_body(vcs_hbm, srcp_hbm, dstp_hbm, zeros_hbm, out_hbm,
             v_sp, acc, src_idx, dst_idx, rows, gsem, ssem):
    cid = lax.axis_index("c")
    sid = lax.axis_index("s")
    rslice = pl.ds(sid * ROWS_PER_TILE, ROWS_PER_TILE)

    pltpu.sync_copy(vcs_hbm.at[cid, rslice], v_sp.at[rslice])
    pltpu.sync_copy(zeros_hbm, acc.at[rslice])
    plsc.subcore_barrier()

    for s in range(NSEG):
        pltpu.sync_copy(srcp_hbm.at[sid, pl.ds(s * SEG_CHUNKS, SEG_CHUNKS)],
                        src_idx)
        pltpu.sync_copy(dstp_hbm.at[sid, pl.ds(s * SEG_CHUNKS, SEG_CHUNKS)],
                        dst_idx)

        def body(j, carry):
            base = j * BLK
            pre = NBUF - 1
            d_g = {}
            d_s = {}
            for i in range(pre):
                d_g[i] = pltpu.async_copy(
                    v_sp.at[src_idx.at[base + i]], rows.at[i % NBUF], gsem)
            for i in range(BLK):
                if i + pre < BLK:
                    if i + pre - NBUF >= 0:
                        d_s[i + pre - NBUF].wait()
                    d_g[i + pre] = pltpu.async_copy(
                        v_sp.at[src_idx.at[base + i + pre]],
                        rows.at[(i + pre) % NBUF], gsem)
                d_g[i].wait()
                d_s[i] = pltpu.async_copy(
                    rows.at[i % NBUF], acc.at[dst_idx.at[base + i]], ssem,
                    add=True)
            for i in range(max(0, BLK - NBUF), BLK):
                d_s[i].wait()
            return carry

        lax.fori_loop(0, SEG_CHUNKS // BLK, body, 0)
    plsc.subcore_barrier()
    pltpu.sync_copy(acc.at[rslice], out_hbm.at[cid, rslice])


_sc_aggregate = functools.partial(
    pl.kernel,
    out_type=jax.ShapeDtypeStruct((NUM_CORES, N_ROWS, COLS), jnp.float32),
    mesh=plsc.VectorSubcoreMesh(
        core_axis_name="c", subcore_axis_name="s",
        num_cores=NUM_CORES, num_subcores=NUM_SUBCORES),
    scratch_types=[
        pltpu.VMEM_SHARED((N_ROWS, COLS), jnp.float32),
        pltpu.VMEM_SHARED((N_ROWS, COLS), jnp.float32),
        pltpu.VMEM((SEG_CHUNKS, CHUNK), jnp.int32),
        pltpu.VMEM((SEG_CHUNKS, CHUNK), jnp.int32),
        pltpu.VMEM((NBUF, CHUNK, COLS), jnp.float32),
        pltpu.SemaphoreType.DMA,
        pltpu.SemaphoreType.DMA,
    ],
)(_sc_body)


def _tc_body(p_ref, v_ref, eps_ref, W1_ref, b1_ref, g1_ref, be1_ref,
             W2_ref, b2_ref, g2_ref, be2_ref, out_ref):
    eps = eps_ref[0, 0]
    x = jnp.concatenate([p_ref[0, :N, :], p_ref[1, :N, :]], axis=1)
    x = x + eps * v_ref[...]

    h = lax.dot_general(x, W1_ref[...], (((1,), (1,)), ((), ())),
                        preferred_element_type=jnp.float32) + b1_ref[...]
    mean = jnp.mean(h, axis=0, keepdims=True)
    var = jnp.mean((h - mean) * (h - mean), axis=0, keepdims=True)
    h = (h - mean) * lax.rsqrt(var + BN_EPS) * g1_ref[...] + be1_ref[...]
    h = jnp.maximum(h, 0.0)

    h = lax.dot_general(h, W2_ref[...], (((1,), (1,)), ((), ())),
                        preferred_element_type=jnp.float32) + b2_ref[...]
    mean = jnp.mean(h, axis=0, keepdims=True)
    var = jnp.mean((h - mean) * (h - mean), axis=0, keepdims=True)
    h = (h - mean) * lax.rsqrt(var + BN_EPS) * g2_ref[...] + be2_ref[...]
    out_ref[...] = jnp.maximum(h, 0.0)


def kernel(v, a, epsilon, W1, b1, g1, be1, W2, b2, g2, be2):
    src = a[0].astype(jnp.int32)
    dst = a[1].astype(jnp.int32)
    pad = E_PAD - E
    spread = N + (jnp.arange(pad, dtype=jnp.int32) % (N_ROWS - N))
    srcp = jnp.concatenate([src, spread])
    dstp = jnp.concatenate([dst, spread])
    srcp = srcp.reshape(NUM_SUBCORES, CHUNKS_PER_T, CHUNK)
    dstp = dstp.reshape(NUM_SUBCORES, CHUNKS_PER_T, CHUNK)
    vp = jnp.concatenate([v, jnp.zeros((N_ROWS - N, D), jnp.float32)])
    vcs = jnp.stack([vp[:, :COLS], vp[:, COLS:]])
    zeros_blk = jnp.zeros((ROWS_PER_TILE, COLS), jnp.float32)

    parts = _sc_aggregate(vcs, srcp, dstp, zeros_blk)

    out = pl.pallas_call(
        _tc_body,
        out_shape=jax.ShapeDtypeStruct((N, D), jnp.float32),
    )(parts, v, epsilon,
      W1, b1.reshape(1, D), g1.reshape(1, D), be1.reshape(1, D),
      W2, b2.reshape(1, D), g2.reshape(1, D), be2.reshape(1, D))
    return out


# speedup vs baseline: 1.2831x; 1.0179x over previous
import functools

import jax
import jax.numpy as jnp
from jax import lax
from jax.experimental import pallas as pl
from jax.experimental.pallas import tpu as pltpu
from jax.experimental.pallas import tpu_sc as plsc

N = 10000
E = 320000
D = 128
BN_EPS = 1e-5

NUM_CORES = 2
NUM_SUBCORES = 16
COLS = D // NUM_CORES
CHUNK = 128
CHUNKS_PER_T = 160
NSEG = 5
SEG_CHUNKS = CHUNKS_PER_T // NSEG
BLK = 8
NBUF = 2
E_PAD = NUM_SUBCORES * CHUNK * CHUNKS_PER_T
N_ROWS = 10112
ROWS_PER_TILE = N_ROWS // NUM_SUBCORES


def _sc_body(vcs_hbm, srcp_hbm, dstp_hbm, zeros_hbm, out_hbm,
             v_sp, acc, src_idx, dst_idx, rows, gsem, ssem, isem):
    cid = lax.axis_index("c")
    sid = lax.axis_index("s")
    rslice = pl.ds(sid * ROWS_PER_TILE, ROWS_PER_TILE)

    pltpu.sync_copy(vcs_hbm.at[cid, rslice], v_sp.at[rslice])
    pltpu.sync_copy(zeros_hbm, acc.at[rslice])
    plsc.subcore_barrier()

    d_i = {}

    def stage(s):
        bank = s % 2
        d_i[s] = (
            pltpu.async_copy(
                srcp_hbm.at[sid, pl.ds(s * SEG_CHUNKS, SEG_CHUNKS)],
                src_idx.at[bank], isem),
            pltpu.async_copy(
                dstp_hbm.at[sid, pl.ds(s * SEG_CHUNKS, SEG_CHUNKS)],
                dst_idx.at[bank], isem),
        )

    stage(0)
    for s in range(NSEG):
        for d in d_i.pop(s):
            d.wait()
        if s + 1 < NSEG:
            stage(s + 1)
        bank = s % 2
        src_seg = src_idx.at[bank]
        dst_seg = dst_idx.at[bank]

        def body(j, carry):
            base = j * BLK
            pre = NBUF - 1

            def buf(i):
                return rows.at[i % NBUF]

            d_g = {}
            d_s = {}
            for i in range(pre):
                d_g[i] = pltpu.async_copy(
                    v_sp.at[src_seg.at[base + i]], buf(i), gsem)
            for i in range(BLK):
                if i + pre < BLK:
                    if i + pre - NBUF >= 0:
                        d_s[i + pre - NBUF].wait()
                    d_g[i + pre] = pltpu.async_copy(
                        v_sp.at[src_seg.at[base + i + pre]],
                        buf(i + pre), gsem)
                d_g[i].wait()
                d_s[i] = pltpu.async_copy(
                    buf(i), acc.at[dst_seg.at[base + i]], ssem,
                    add=True)
            for i in range(max(0, BLK - NBUF), BLK):
                d_s[i].wait()
            return carry

        lax.fori_loop(0, SEG_CHUNKS // BLK, body, 0)
    plsc.subcore_barrier()
    pltpu.sync_copy(acc.at[rslice], out_hbm.at[cid, rslice])


_sc_aggregate = functools.partial(
    pl.kernel,
    out_type=jax.ShapeDtypeStruct((NUM_CORES, N_ROWS, COLS), jnp.float32),
    mesh=plsc.VectorSubcoreMesh(
        core_axis_name="c", subcore_axis_name="s",
        num_cores=NUM_CORES, num_subcores=NUM_SUBCORES),
    scratch_types=[
        pltpu.VMEM_SHARED((N_ROWS, COLS), jnp.float32),
        pltpu.VMEM_SHARED((N_ROWS, COLS), jnp.float32),
        pltpu.VMEM((2, SEG_CHUNKS, CHUNK), jnp.int32),
        pltpu.VMEM((2, SEG_CHUNKS, CHUNK), jnp.int32),
        pltpu.VMEM((NBUF, CHUNK, COLS), jnp.float32),
        pltpu.SemaphoreType.DMA,
        pltpu.SemaphoreType.DMA,
        pltpu.SemaphoreType.DMA,
    ],
)(_sc_body)


def _tc_body(p_ref, v_ref, eps_ref, W1_ref, b1_ref, g1_ref, be1_ref,
             W2_ref, b2_ref, g2_ref, be2_ref, out_ref):
    eps = eps_ref[0, 0]
    x = jnp.concatenate([p_ref[0, :N, :], p_ref[1, :N, :]], axis=1)
    x = x + eps * v_ref[...]

    h = lax.dot_general(x, W1_ref[...], (((1,), (1,)), ((), ())),
                        preferred_element_type=jnp.float32) + b1_ref[...]
    mean = jnp.mean(h, axis=0, keepdims=True)
    var = jnp.mean((h - mean) * (h - mean), axis=0, keepdims=True)
    h = (h - mean) * lax.rsqrt(var + BN_EPS) * g1_ref[...] + be1_ref[...]
    h = jnp.maximum(h, 0.0)

    h = lax.dot_general(h, W2_ref[...], (((1,), (1,)), ((), ())),
                        preferred_element_type=jnp.float32) + b2_ref[...]
    mean = jnp.mean(h, axis=0, keepdims=True)
    var = jnp.mean((h - mean) * (h - mean), axis=0, keepdims=True)
    h = (h - mean) * lax.rsqrt(var + BN_EPS) * g2_ref[...] + be2_ref[...]
    out_ref[...] = jnp.maximum(h, 0.0)


def kernel(v, a, epsilon, W1, b1, g1, be1, W2, b2, g2, be2):
    src = a[0].astype(jnp.int32)
    dst = a[1].astype(jnp.int32)
    pad = E_PAD - E
    spread = N + (jnp.arange(pad, dtype=jnp.int32) % (N_ROWS - N))
    srcp = jnp.concatenate([src, spread])
    dstp = jnp.concatenate([dst, spread])
    srcp = srcp.reshape(NUM_SUBCORES, CHUNKS_PER_T, CHUNK)
    dstp = dstp.reshape(NUM_SUBCORES, CHUNKS_PER_T, CHUNK)
    vp = jnp.concatenate([v, jnp.zeros((N_ROWS - N, D), jnp.float32)])
    vcs = jnp.stack([vp[:, :COLS], vp[:, COLS:]])
    zeros_blk = jnp.zeros((ROWS_PER_TILE, COLS), jnp.float32)

    parts = _sc_aggregate(vcs, srcp, dstp, zeros_blk)

    out = pl.pallas_call(
        _tc_body,
        out_shape=jax.ShapeDtypeStruct((N, D), jnp.float32),
    )(parts, v, epsilon,
      W1, b1.reshape(1, D), g1.reshape(1, D), be1.reshape(1, D),
      W2, b2.reshape(1, D), g2.reshape(1, D), be2.reshape(1, D))
    return out
